# baseline (speedup 1.0000x reference)
import functools

import jax
import jax.numpy as jnp
from jax import lax
from jax.experimental import pallas as pl
from jax.experimental.pallas import tpu as pltpu
from jax.experimental.pallas import tpu_sc as plsc

L = 16
NC = 2
NS = 16
NW = NC * NS
CHUNK = 128


def _round_up(a, b):
    return (a + b - 1) // b * b


def _sc_mesh():
    return plsc.VectorSubcoreMesh(core_axis_name="c", subcore_axis_name="s")


def _deg_body(n_pad, nch, f, dst_hbm, ones_hbm, zer_hbm, degp_hbm,
              didx_all, ones_v, acc_sh, sem):
    c = lax.axis_index("c")
    s = lax.axis_index("s")
    wid = s * NC + c
    r_pt = n_pad // NS
    pltpu.sync_copy(zer_hbm, acc_sh.at[pl.ds(s * r_pt, r_pt)])
    pltpu.sync_copy(ones_hbm, ones_v)
    pltpu.sync_copy(dst_hbm.at[wid], didx_all)
    plsc.subcore_barrier()

    k = 4

    def gbody(g, carry):
        base = g * k
        descs = [
            pltpu.async_copy(ones_v, acc_sh.at[didx_all.at[base + b]], sem,
                             add=True)
            for b in range(k)
        ]
        for d in descs:
            d.wait()
        return carry

    lax.fori_loop(0, nch // k, gbody, 0)
    plsc.subcore_barrier()
    pltpu.sync_copy(acc_sh.at[pl.ds(s * r_pt, r_pt)],
                    degp_hbm.at[c, pl.ds(s * r_pt, r_pt)])


def _make_deg_call(n_pad, nch, f):
    return pl.kernel(
        functools.partial(_deg_body, n_pad, nch, f),
        out_type=jax.ShapeDtypeStruct((NC, n_pad, f), jnp.float32),
        mesh=_sc_mesh(),
        scratch_types=[
            pltpu.VMEM((nch, CHUNK), jnp.int32),
            pltpu.VMEM((CHUNK, f), jnp.float32),
            pltpu.VMEM_SHARED((n_pad, f), jnp.float32),
            pltpu.SemaphoreType.DMA,
        ],
    )


def _scat_body(n_pad, nch, f, h_hbm, src_hbm, dst_hbm, zer_hbm, parts_hbm,
               sidx_all, didx, rows, acc_sh, sem):
    c = lax.axis_index("c")
    s = lax.axis_index("s")
    wid = s * NC + c
    r_pt = n_pad // NS
    pltpu.sync_copy(zer_hbm, acc_sh.at[pl.ds(s * r_pt, r_pt)])
    pltpu.sync_copy(src_hbm.at[wid], sidx_all)
    plsc.subcore_barrier()

    def ebody(j, carry):
        pltpu.sync_copy(dst_hbm.at[wid, j], didx)
        pltpu.async_copy(h_hbm.at[sidx_all.at[j]], rows, sem).wait()
        pltpu.sync_copy(rows, acc_sh.at[didx], add=True)
        return carry

    lax.fori_loop(0, nch, ebody, 0)
    plsc.subcore_barrier()
    pltpu.sync_copy(acc_sh.at[pl.ds(s * r_pt, r_pt)],
                    parts_hbm.at[c, pl.ds(s * r_pt, r_pt)])


def _make_scat_call(n_pad, nch, f):
    return pl.kernel(
        functools.partial(_scat_body, n_pad, nch, f),
        out_type=jax.ShapeDtypeStruct((NC, n_pad, f), jnp.float32),
        mesh=_sc_mesh(),
        scratch_types=[
            pltpu.VMEM((nch, CHUNK), jnp.int32),
            pltpu.VMEM((CHUNK,), jnp.int32),
            pltpu.VMEM((CHUNK, f), jnp.float32),
            pltpu.VMEM_SHARED((n_pad, f), jnp.float32),
            pltpu.SemaphoreType.DMA,
        ],
    )


def _dinv_body(degp_ref, dinv_ref):
    p = degp_ref[...]
    deg = p[0, :, 0:1] + p[1, :, 0:1] + 1.0
    dinv_ref[...] = lax.rsqrt(deg)


def _h1_body(x_ref, w_ref, dv_ref, o_ref):
    h = jnp.dot(x_ref[...], w_ref[...], preferred_element_type=jnp.float32)
    o_ref[...] = h * dv_ref[...]


def _mid_body(p_ref, h_ref, dv_ref, b_ref, w_ref, o_ref):
    acc = p_ref[0] + p_ref[1] + h_ref[...]
    x2 = jnp.maximum(acc * dv_ref[...] + b_ref[...], 0.0)
    h2 = jnp.dot(x2, w_ref[...], preferred_element_type=jnp.float32)
    o_ref[...] = h2 * dv_ref[...]


def _fin_body(num_graphs, p_ref, h_ref, dv_ref, b_ref, bt_ref, wl_ref,
              bl_ref, o_ref, pool_acc, cnt_acc):
    i = pl.program_id(0)

    @pl.when(i == 0)
    def _():
        pool_acc[...] = jnp.zeros_like(pool_acc)
        cnt_acc[...] = jnp.zeros_like(cnt_acc)

    acc = p_ref[0] + p_ref[1] + h_ref[...]
    h2 = jnp.maximum(acc * dv_ref[...] + b_ref[...], 0.0)
    gids = lax.broadcasted_iota(jnp.int32, (num_graphs, 1), 0)
    onehot = (gids == bt_ref[0:1, :]).astype(jnp.float32)
    pool_acc[...] += jnp.dot(onehot, h2, preferred_element_type=jnp.float32)
    cnt_acc[...] += jnp.sum(onehot, axis=1, keepdims=True)

    @pl.when(i == pl.num_programs(0) - 1)
    def _():
        g = pool_acc[...] / jnp.maximum(cnt_acc[...], 1.0)
        o_ref[...] = (
            jnp.dot(g, wl_ref[...], preferred_element_type=jnp.float32)
            + bl_ref[...]
        )


def kernel(x, edge_index, batch, W1, b1, W2, b2, Wlin, blin):
    n, f_in = x.shape
    e = edge_index.shape[1]
    h = W1.shape[1]
    num_graphs = 64

    n_pad = _round_up(n + 1, 512)
    nch = _round_up(-(-e // (NW * CHUNK)), 8)
    e_pad = NW * nch * CHUNK
    blk = n_pad // 4

    src3 = jnp.concatenate(
        [edge_index[0], jnp.zeros((e_pad - e,), jnp.int32)]
    ).reshape(NW, nch, CHUNK)
    dst3 = jnp.concatenate(
        [edge_index[1], jnp.full((e_pad - e,), n, jnp.int32)]
    ).reshape(NW, nch, CHUNK)
    x_p = jnp.concatenate([x, jnp.zeros((n_pad - n, f_in), x.dtype)])
    batch_p = jnp.concatenate(
        [batch, jnp.full((n_pad - n,), num_graphs, jnp.int32)])
    batch2d = jnp.broadcast_to(batch_p[None, :], (8, n_pad))
    zer = jnp.zeros((n_pad // NS, h), jnp.float32)
    ones_rows = jnp.ones((CHUNK, h), jnp.float32)
    b1r = b1.reshape(1, h)
    b2r = b2.reshape(1, h)
    wl_p = jnp.pad(Wlin, ((0, 0), (0, h - Wlin.shape[1])))
    bl_p = jnp.pad(blin, (0, h - blin.shape[0])).reshape(1, h)

    degp = _make_deg_call(n_pad, nch, h)(dst3, ones_rows, zer)

    dinv2 = pl.pallas_call(
        _dinv_body,
        out_shape=jax.ShapeDtypeStruct((n_pad, 1), jnp.float32),
    )(degp)

    grid = n_pad // blk
    h1p = pl.pallas_call(
        _h1_body,
        grid=(grid,),
        in_specs=[
            pl.BlockSpec((blk, f_in), lambda i: (i, 0)),
            pl.BlockSpec((f_in, h), lambda i: (0, 0)),
            pl.BlockSpec((blk, 1), lambda i: (i, 0)),
        ],
        out_specs=pl.BlockSpec((blk, h), lambda i: (i, 0)),
        out_shape=jax.ShapeDtypeStruct((n_pad, h), jnp.float32),
    )(x_p, W1, dinv2)

    scat = _make_scat_call(n_pad, nch, h)

    parts1 = scat(h1p, src3, dst3, zer)

    h2p = pl.pallas_call(
        _mid_body,
        grid=(grid,),
        in_specs=[
            pl.BlockSpec((NC, blk, h), lambda i: (0, i, 0)),
            pl.BlockSpec((blk, h), lambda i: (i, 0)),
            pl.BlockSpec((blk, 1), lambda i: (i, 0)),
            pl.BlockSpec((1, h), lambda i: (0, 0)),
            pl.BlockSpec((h, h), lambda i: (0, 0)),
        ],
        out_specs=pl.BlockSpec((blk, h), lambda i: (i, 0)),
        out_shape=jax.ShapeDtypeStruct((n_pad, h), jnp.float32),
    )(parts1, h1p, dinv2, b1r, W2)

    parts2 = scat(h2p, src3, dst3, zer)

    out_pad = pl.pallas_call(
        functools.partial(_fin_body, num_graphs),
        grid=(grid,),
        in_specs=[
            pl.BlockSpec((NC, blk, h), lambda i: (0, i, 0)),
            pl.BlockSpec((blk, h), lambda i: (i, 0)),
            pl.BlockSpec((blk, 1), lambda i: (i, 0)),
            pl.BlockSpec((1, h), lambda i: (0, 0)),
            pl.BlockSpec((8, blk), lambda i: (0, i)),
            pl.BlockSpec((h, h), lambda i: (0, 0)),
            pl.BlockSpec((1, h), lambda i: (0, 0)),
        ],
        out_specs=pl.BlockSpec((num_graphs, h), lambda i: (0, 0)),
        out_shape=jax.ShapeDtypeStruct((num_graphs, h), jnp.float32),
        scratch_shapes=[
            pltpu.VMEM((num_graphs, h), jnp.float32),
            pltpu.VMEM((num_graphs, 1), jnp.float32),
        ],
    )(parts2, h2p, dinv2, b2r, batch2d, wl_p, bl_p)

    return out_pad[:, : Wlin.shape[1]]


# speedup vs baseline: 1.3161x; 1.3161x over previous
import functools

import jax
import jax.numpy as jnp
from jax import lax
from jax.experimental import pallas as pl
from jax.experimental.pallas import tpu as pltpu
from jax.experimental.pallas import tpu_sc as plsc

L = 16
NC = 2
NS = 16
NW = NC * NS
CHUNK = 128


def _round_up(a, b):
    return (a + b - 1) // b * b


def _sc_mesh():
    return plsc.VectorSubcoreMesh(core_axis_name="c", subcore_axis_name="s")


def _deg_body(n_pad, nch, f, dst_hbm, ones_hbm, zer_hbm, degp_hbm,
              didx_all, ones_v, acc_sh, sem):
    c = lax.axis_index("c")
    s = lax.axis_index("s")
    wid = s * NC + c
    r_pt = n_pad // NS
    pltpu.sync_copy(zer_hbm, acc_sh.at[pl.ds(s * r_pt, r_pt)])
    pltpu.sync_copy(ones_hbm, ones_v)
    pltpu.sync_copy(dst_hbm.at[wid], didx_all)
    plsc.subcore_barrier()

    k = 4

    def gbody(g, carry):
        base = g * k
        descs = [
            pltpu.async_copy(ones_v, acc_sh.at[didx_all.at[base + b]], sem,
                             add=True)
            for b in range(k)
        ]
        for d in descs:
            d.wait()
        return carry

    lax.fori_loop(0, nch // k, gbody, 0)
    plsc.subcore_barrier()
    pltpu.sync_copy(acc_sh.at[pl.ds(s * r_pt, r_pt)],
                    degp_hbm.at[c, pl.ds(s * r_pt, r_pt)])


def _make_deg_call(n_pad, nch, f):
    return pl.kernel(
        functools.partial(_deg_body, n_pad, nch, f),
        out_type=jax.ShapeDtypeStruct((NC, n_pad, f), jnp.float32),
        mesh=_sc_mesh(),
        scratch_types=[
            pltpu.VMEM((nch, CHUNK), jnp.int32),
            pltpu.VMEM((CHUNK, f), jnp.float32),
            pltpu.VMEM_SHARED((n_pad, f), jnp.float32),
            pltpu.SemaphoreType.DMA,
        ],
    )


def _scat_body(n_pad, e_pt, f, h_hbm, src_hbm, dst_hbm, zer_hbm, parts_hbm,
               sidx, didx, rows, acc_sh, sem):
    c = lax.axis_index("c")
    s = lax.axis_index("s")
    wid = s * NC + c
    r_pt = n_pad // NS
    pltpu.sync_copy(zer_hbm, acc_sh.at[pl.ds(s * r_pt, r_pt)])
    plsc.subcore_barrier()

    base = wid * e_pt

    def ebody(j, carry):
        off = base + j * CHUNK
        pltpu.sync_copy(src_hbm.at[pl.ds(off, CHUNK)], sidx)
        pltpu.sync_copy(dst_hbm.at[pl.ds(off, CHUNK)], didx)
        pltpu.async_copy(h_hbm.at[sidx], rows, sem).wait()
        pltpu.sync_copy(rows, acc_sh.at[didx], add=True)
        return carry

    lax.fori_loop(0, e_pt // CHUNK, ebody, 0)
    plsc.subcore_barrier()
    pltpu.sync_copy(acc_sh.at[pl.ds(s * r_pt, r_pt)],
                    parts_hbm.at[c, pl.ds(s * r_pt, r_pt)])


def _make_scat_call(n_pad, e_pad, f):
    e_pt = e_pad // NW
    return pl.kernel(
        functools.partial(_scat_body, n_pad, e_pt, f),
        out_type=jax.ShapeDtypeStruct((NC, n_pad, f), jnp.float32),
        mesh=_sc_mesh(),
        scratch_types=[
            pltpu.VMEM((CHUNK,), jnp.int32),
            pltpu.VMEM((CHUNK,), jnp.int32),
            pltpu.VMEM((CHUNK, f), jnp.float32),
            pltpu.VMEM_SHARED((n_pad, f), jnp.float32),
            pltpu.SemaphoreType.DMA,
        ],
    )


def _dinv_body(degp_ref, dinv_ref):
    p = degp_ref[...]
    deg = p[0, :, 0:1] + p[1, :, 0:1] + 1.0
    dinv_ref[...] = lax.rsqrt(deg)


def _h1_body(x_ref, w_ref, dv_ref, o_ref):
    h = jnp.dot(x_ref[...], w_ref[...], preferred_element_type=jnp.float32)
    o_ref[...] = h * dv_ref[...]


def _mid_body(p_ref, h_ref, dv_ref, b_ref, w_ref, o_ref):
    acc = p_ref[0] + p_ref[1] + h_ref[...]
    x2 = jnp.maximum(acc * dv_ref[...] + b_ref[...], 0.0)
    h2 = jnp.dot(x2, w_ref[...], preferred_element_type=jnp.float32)
    o_ref[...] = h2 * dv_ref[...]


def _fin_body(num_graphs, p_ref, h_ref, dv_ref, b_ref, bt_ref, wl_ref,
              bl_ref, o_ref, pool_acc, cnt_acc):
    i = pl.program_id(0)

    @pl.when(i == 0)
    def _():
        pool_acc[...] = jnp.zeros_like(pool_acc)
        cnt_acc[...] = jnp.zeros_like(cnt_acc)

    acc = p_ref[0] + p_ref[1] + h_ref[...]
    h2 = jnp.maximum(acc * dv_ref[...] + b_ref[...], 0.0)
    gids = lax.broadcasted_iota(jnp.int32, (num_graphs, 1), 0)
    onehot = (gids == bt_ref[0:1, :]).astype(jnp.float32)
    pool_acc[...] += jnp.dot(onehot, h2, preferred_element_type=jnp.float32)
    cnt_acc[...] += jnp.sum(onehot, axis=1, keepdims=True)

    @pl.when(i == pl.num_programs(0) - 1)
    def _():
        g = pool_acc[...] / jnp.maximum(cnt_acc[...], 1.0)
        o_ref[...] = (
            jnp.dot(g, wl_ref[...], preferred_element_type=jnp.float32)
            + bl_ref[...]
        )


def kernel(x, edge_index, batch, W1, b1, W2, b2, Wlin, blin):
    n, f_in = x.shape
    e = edge_index.shape[1]
    h = W1.shape[1]
    num_graphs = 64

    n_pad = _round_up(n + 1, 512)
    e_pt = _round_up(-(-e // NW), CHUNK)
    e_pad = e_pt * NW
    nch_deg = _round_up(-(-e // (NW * CHUNK)), 4)
    e_deg = NW * nch_deg * CHUNK
    blk = n_pad // 4

    src = jnp.concatenate(
        [edge_index[0], jnp.zeros((e_pad - e,), jnp.int32)])
    dst = jnp.concatenate(
        [edge_index[1], jnp.full((e_pad - e,), n, jnp.int32)])
    dst3 = jnp.concatenate(
        [edge_index[1], jnp.full((e_deg - e,), n, jnp.int32)]
    ).reshape(NW, nch_deg, CHUNK)
    x_p = jnp.concatenate([x, jnp.zeros((n_pad - n, f_in), x.dtype)])
    batch_p = jnp.concatenate(
        [batch, jnp.full((n_pad - n,), num_graphs, jnp.int32)])
    batch2d = jnp.broadcast_to(batch_p[None, :], (8, n_pad))
    zer = jnp.zeros((n_pad // NS, h), jnp.float32)
    ones_rows = jnp.ones((CHUNK, h), jnp.float32)
    b1r = b1.reshape(1, h)
    b2r = b2.reshape(1, h)
    wl_p = jnp.pad(Wlin, ((0, 0), (0, h - Wlin.shape[1])))
    bl_p = jnp.pad(blin, (0, h - blin.shape[0])).reshape(1, h)

    degp = _make_deg_call(n_pad, nch_deg, h)(dst3, ones_rows, zer)

    dinv2 = pl.pallas_call(
        _dinv_body,
        out_shape=jax.ShapeDtypeStruct((n_pad, 1), jnp.float32),
    )(degp)

    grid = n_pad // blk
    h1p = pl.pallas_call(
        _h1_body,
        grid=(grid,),
        in_specs=[
            pl.BlockSpec((blk, f_in), lambda i: (i, 0)),
            pl.BlockSpec((f_in, h), lambda i: (0, 0)),
            pl.BlockSpec((blk, 1), lambda i: (i, 0)),
        ],
        out_specs=pl.BlockSpec((blk, h), lambda i: (i, 0)),
        out_shape=jax.ShapeDtypeStruct((n_pad, h), jnp.float32),
    )(x_p, W1, dinv2)

    scat = _make_scat_call(n_pad, e_pad, h)

    parts1 = scat(h1p, src, dst, zer)

    h2p = pl.pallas_call(
        _mid_body,
        grid=(grid,),
        in_specs=[
            pl.BlockSpec((NC, blk, h), lambda i: (0, i, 0)),
            pl.BlockSpec((blk, h), lambda i: (i, 0)),
            pl.BlockSpec((blk, 1), lambda i: (i, 0)),
            pl.BlockSpec((1, h), lambda i: (0, 0)),
            pl.BlockSpec((h, h), lambda i: (0, 0)),
        ],
        out_specs=pl.BlockSpec((blk, h), lambda i: (i, 0)),
        out_shape=jax.ShapeDtypeStruct((n_pad, h), jnp.float32),
    )(parts1, h1p, dinv2, b1r, W2)

    parts2 = scat(h2p, src, dst, zer)

    out_pad = pl.pallas_call(
        functools.partial(_fin_body, num_graphs),
        grid=(grid,),
        in_specs=[
            pl.BlockSpec((NC, blk, h), lambda i: (0, i, 0)),
            pl.BlockSpec((blk, h), lambda i: (i, 0)),
            pl.BlockSpec((blk, 1), lambda i: (i, 0)),
            pl.BlockSpec((1, h), lambda i: (0, 0)),
            pl.BlockSpec((8, blk), lambda i: (0, i)),
            pl.BlockSpec((h, h), lambda i: (0, 0)),
            pl.BlockSpec((1, h), lambda i: (0, 0)),
        ],
        out_specs=pl.BlockSpec((num_graphs, h), lambda i: (0, 0)),
        out_shape=jax.ShapeDtypeStruct((num_graphs, h), jnp.float32),
        scratch_shapes=[
            pltpu.VMEM((num_graphs, h), jnp.float32),
            pltpu.VMEM((num_graphs, 1), jnp.float32),
        ],
    )(parts2, h2p, dinv2, b2r, batch2d, wl_p, bl_p)

    return out_pad[:, : Wlin.shape[1]]
